# baseline (speedup 1.0000x reference)
import functools

import jax
import jax.numpy as jnp
from jax import lax
from jax.experimental import pallas as pl
from jax.experimental.pallas import tpu as pltpu
from jax.experimental.pallas import tpu_sc as plsc

_N = 10000
_D = 128
_E = 650000
_NC, _NS, _L = 2, 16, 16
_NW = _NC * _NS
_CHUNK = 96
_BLK = 4
_CPT = 424
_EPAD = _NS * _CHUNK * _CPT
_HALF = 5120
_AROWS = 5248
_RPT = _AROWS // _NS
_XROWS = 10112

_ROWS = 1000


def _mm_body(h_ref, w_ref, b_ref, o_ref, *, act):
    y = jnp.dot(h_ref[...], w_ref[...], preferred_element_type=jnp.float32)
    y = y + b_ref[...][None, :]
    if act == "relu":
        y = jnp.maximum(y, 0.0)
    o_ref[...] = y


def _mm(h, W, b, act=None):
    N, K = h.shape
    M = W.shape[1]
    return pl.pallas_call(
        functools.partial(_mm_body, act=act),
        grid=(N // _ROWS,),
        in_specs=[
            pl.BlockSpec((_ROWS, K), lambda i: (i, 0)),
            pl.BlockSpec((K, M), lambda i: (0, 0)),
            pl.BlockSpec((M,), lambda i: (0,)),
        ],
        out_specs=pl.BlockSpec((_ROWS, M), lambda i: (i, 0)),
        out_shape=jax.ShapeDtypeStruct((N, M), jnp.float32),
    )(h, W, b)


def _combine_body(a_ref, d_ref, bias_ref, res_ref, o_ref):
    den = jnp.sum(d_ref[...], axis=1)
    gat = a_ref[...] / (den + 1e-16)[:, None] + bias_ref[...][None, :]
    o_ref[...] = jnp.maximum(gat, 0.0) + res_ref[...]


def _combine(msg, den, bias, res):
    return pl.pallas_call(
        _combine_body,
        grid=(_N // _ROWS,),
        in_specs=[
            pl.BlockSpec((_ROWS, _D), lambda i: (i, 0)),
            pl.BlockSpec((_ROWS, _NS), lambda i: (i, 0)),
            pl.BlockSpec((_D,), lambda i: (0,)),
            pl.BlockSpec((_ROWS, _D), lambda i: (i, 0)),
        ],
        out_specs=pl.BlockSpec((_ROWS, _D), lambda i: (i, 0)),
        out_shape=jax.ShapeDtypeStruct((_N, _D), jnp.float32),
    )(msg, den, bias, res)


_CPT1 = _EPAD // (_NW * _CHUNK)


def _w_body(xl_hbm, xr_hbm, src_hbm, dst_hbm, att_hbm, w_hbm,
            sb_v, db_v, xl0_v, xl1_v, xr0_v, xr1_v, wbuf_v, att_v,
            sem0, sem1):
    cid = lax.axis_index("c")
    sid = lax.axis_index("s")
    wid = sid * _NC + cid

    zeros16 = jnp.zeros((_L,), jnp.float32)
    xl_b, xr_b = (xl0_v, xl1_v), (xr0_v, xr1_v)
    sem_b = (sem0, sem1)

    pltpu.sync_copy(att_hbm, att_v)
    att_r = [att_v[pl.ds(j * _L, _L)] for j in range(_D // _L)]
    iota16 = lax.iota(jnp.int32, _L)

    def _issue(c):
        p = c % 2
        sl = pl.ds(c * _CHUNK, _CHUNK)
        return (
            pltpu.async_copy(xl_hbm.at[sb_v.at[sl]], xl_b[p], sem_b[p]),
            pltpu.async_copy(xr_hbm.at[db_v.at[sl]], xr_b[p], sem_b[p]),
        )

    def _compute(c, e0):
        p = c % 2
        xl_v, xr_v = xl_b[p], xr_b[p]

        def _group(g, c2):
            w_vec = zeros16
            for k in range(_L):
                e = g * _L + k
                terms = []
                for j in range(_D // _L):
                    sl = pl.ds(j * _L, _L)
                    v = xl_v[e, sl] + xr_v[e, sl]
                    terms.append(jnp.maximum(v, 0.2 * v) * att_r[j])
                while len(terms) > 1:
                    terms = [a + b for a, b in zip(terms[::2], terms[1::2])]
                alpha = jnp.sum(terms[0])
                w_spl = jnp.exp(jnp.broadcast_to(alpha, (_L,)))
                w_vec = jnp.where(iota16 == k, w_spl, w_vec)
            wbuf_v[pl.ds(g * _L, _L)] = w_vec
            return c2
        lax.fori_loop(0, _CHUNK // _L, _group, 0)
        pltpu.sync_copy(wbuf_v, w_hbm.at[pl.ds(e0 + c * _CHUNK, _CHUNK)])

    def _block(b, carry):
        e0 = (wid * _CPT1 + b * _BLK) * _CHUNK
        pltpu.sync_copy(src_hbm.at[pl.ds(e0, _BLK * _CHUNK)], sb_v)
        pltpu.sync_copy(dst_hbm.at[pl.ds(e0, _BLK * _CHUNK)], db_v)
        cps = _issue(0)
        for c in range(_BLK):
            nxt = _issue(c + 1) if c + 1 < _BLK else None
            for cp in cps:
                cp.wait()
            _compute(c, e0)
            cps = nxt
        return carry
    lax.fori_loop(0, _CPT1 // _BLK, _block, 0)


_w_kernel = pl.kernel(
    _w_body,
    out_type=jax.ShapeDtypeStruct((_EPAD,), jnp.float32),
    mesh=plsc.VectorSubcoreMesh(core_axis_name="c", subcore_axis_name="s",
                                num_cores=_NC, num_subcores=_NS),
    scratch_types=[
        pltpu.VMEM((_BLK * _CHUNK,), jnp.int32),
        pltpu.VMEM((_BLK * _CHUNK,), jnp.int32),
        pltpu.VMEM((_CHUNK, _D), jnp.float32),
        pltpu.VMEM((_CHUNK, _D), jnp.float32),
        pltpu.VMEM((_CHUNK, _D), jnp.float32),
        pltpu.VMEM((_CHUNK, _D), jnp.float32),
        pltpu.VMEM((_CHUNK,), jnp.float32),
        pltpu.VMEM((_D,), jnp.float32),
        pltpu.SemaphoreType.DMA,
        pltpu.SemaphoreType.DMA,
    ],
    compiler_params=pltpu.CompilerParams(needs_layout_passes=False),
)


def _edge_body(xl_hbm, src_hbm, dst_hbm, w_hbm, out_hbm, den_hbm,
               sb_v, db_v, wb_v, dloc_v, xl0_v, xl1_v,
               stg_v, den_v, acc_sh, sem0, sem1):
    cid = lax.axis_index("c")
    sid = lax.axis_index("s")

    zeros16 = jnp.zeros((_L,), jnp.float32)
    nbase = cid * _HALF

    xl_b = (xl0_v, xl1_v)
    sem_b = (sem0, sem1)

    def _zero_row(r, carry):
        for j in range(_D // _L):
            stg_v[r, pl.ds(j * _L, _L)] = zeros16
        return carry
    lax.fori_loop(0, _CHUNK, _zero_row, 0)

    def _zero_den(r, carry):
        den_v[pl.ds(r * _L, _L)] = zeros16
        return carry
    lax.fori_loop(0, _AROWS // _L, _zero_den, 0)

    base_r = sid * _RPT
    for k in range(_RPT // _CHUNK):
        pltpu.sync_copy(stg_v, acc_sh.at[pl.ds(base_r + k * _CHUNK, _CHUNK)])
    rem = _RPT % _CHUNK
    if rem:
        pltpu.sync_copy(stg_v.at[pl.ds(0, rem)],
                        acc_sh.at[pl.ds(base_r + (_RPT // _CHUNK) * _CHUNK, rem)])
    plsc.subcore_barrier()

    def _issue(c):
        p = c % 2
        sl = pl.ds(c * _CHUNK, _CHUNK)
        return (pltpu.async_copy(xl_hbm.at[sb_v.at[sl]], xl_b[p], sem_b[p]),)

    def _compute(c):
        p = c % 2
        xl_v = xl_b[p]

        def _group(g, c2):
            base = c * _CHUNK + g * _L
            w16 = wb_v[pl.ds(base, _L)]
            for k in range(_L):
                e = g * _L + k
                idx = jnp.broadcast_to(base + k, (_L,)).astype(jnp.int32)
                wspl = plsc.load_gather(wb_v, [idx])
                for j in range(_D // _L):
                    sl = pl.ds(j * _L, _L)
                    stg_v[e, sl] = xl_v[e, sl] * wspl
            dl16 = db_v[pl.ds(base, _L)] - nbase
            dl16 = jnp.where((dl16 < 0) | (dl16 >= _HALF), _HALF, dl16)
            dloc_v[pl.ds(g * _L, _L)] = dl16
            plsc.addupdate_scatter(den_v, [dl16], w16)
            return c2
        lax.fori_loop(0, _CHUNK // _L, _group, 0)
        pltpu.sync_copy(stg_v, acc_sh.at[dloc_v], add=True)

    def _block(b, carry):
        e0 = (sid * _CPT + b * _BLK) * _CHUNK
        pltpu.sync_copy(src_hbm.at[pl.ds(e0, _BLK * _CHUNK)], sb_v)
        pltpu.sync_copy(dst_hbm.at[pl.ds(e0, _BLK * _CHUNK)], db_v)
        pltpu.sync_copy(w_hbm.at[pl.ds(e0, _BLK * _CHUNK)], wb_v)
        cps = _issue(0)
        for c in range(_BLK):
            nxt = _issue(c + 1) if c + 1 < _BLK else None
            for cp in cps:
                cp.wait()
            _compute(c)
            cps = nxt
        return carry
    lax.fori_loop(0, _CPT // _BLK, _block, 0)

    pltpu.sync_copy(den_v, den_hbm.at[cid, sid])
    plsc.subcore_barrier()
    pltpu.sync_copy(acc_sh.at[pl.ds(base_r, _RPT)],
                    out_hbm.at[cid, pl.ds(base_r, _RPT)])


_edge_kernel = pl.kernel(
    _edge_body,
    out_type=(
        jax.ShapeDtypeStruct((_NC, _AROWS, _D), jnp.float32),
        jax.ShapeDtypeStruct((_NC, _NS, _AROWS), jnp.float32),
    ),
    mesh=plsc.VectorSubcoreMesh(core_axis_name="c", subcore_axis_name="s",
                                num_cores=_NC, num_subcores=_NS),
    scratch_types=[
        pltpu.VMEM((_BLK * _CHUNK,), jnp.int32),
        pltpu.VMEM((_BLK * _CHUNK,), jnp.int32),
        pltpu.VMEM((_BLK * _CHUNK,), jnp.float32),
        pltpu.VMEM((_CHUNK,), jnp.int32),
        pltpu.VMEM((_CHUNK, _D), jnp.float32),
        pltpu.VMEM((_CHUNK, _D), jnp.float32),
        pltpu.VMEM((_CHUNK, _D), jnp.float32),
        pltpu.VMEM((_AROWS,), jnp.float32),
        pltpu.VMEM_SHARED((_AROWS, _D), jnp.float32),
        pltpu.SemaphoreType.DMA,
        pltpu.SemaphoreType.DMA,
    ],
    compiler_params=pltpu.CompilerParams(needs_layout_passes=False),
)


def _gat_layer(h, src_p, dst_p, Wlr, blr, att, bias, res):
    xlr = _mm(h, Wlr, blr)
    xl = xlr[:, :_D]
    xr = jnp.pad(xlr[:, _D:], ((0, _XROWS - _N), (0, 0)))
    wv = _w_kernel(xl, xr, src_p, dst_p, att)
    acc, den = _edge_kernel(xl, src_p, dst_p, wv)
    msg = jnp.concatenate([acc[0, :_HALF], acc[1, :_N - _HALF]], axis=0)
    den_t = jnp.concatenate(
        [
            den[0].T[:_HALF],
            den[1].T[:_N - _HALF],
        ],
        axis=0,
    )
    return _combine(msg, den_t, bias, res)


def kernel(x, edge_index, W_in, b_in, c1_Wl, c1_bl, c1_Wr, c1_br, c1_att, c1_bias, skip_W, skip_b, c2_Wl, c2_bl, c2_Wr, c2_br, c2_att, c2_bias, c3_Wl, c3_bl, c3_Wr, c3_br, c3_att, c3_bias, m1_W, m1_b, m2_W, m2_b, m3_W, m3_b):
    loop = jnp.arange(_N, dtype=edge_index.dtype)
    pad = _EPAD - _E
    src_p = jnp.concatenate([edge_index[0], loop, jnp.zeros((pad,), jnp.int32)])
    dst_p = jnp.concatenate([edge_index[1], loop, jnp.full((pad,), _N, jnp.int32)])

    x0 = _mm(x, W_in, b_in, act="relu")
    res1 = _mm(x0, skip_W, skip_b)

    Wlr_s = jnp.stack([jnp.concatenate([Wl, Wr], axis=1)
                       for Wl, Wr in ((c1_Wl, c1_Wr), (c2_Wl, c2_Wr), (c3_Wl, c3_Wr))])
    blr_s = jnp.stack([jnp.concatenate([bl, br], axis=0)
                       for bl, br in ((c1_bl, c1_br), (c2_bl, c2_br), (c3_bl, c3_br))])
    att_s = jnp.stack([c1_att[0], c2_att[0], c3_att[0]])
    bias_s = jnp.stack([c1_bias, c2_bias, c3_bias])
    first_s = jnp.array([1.0, 0.0, 0.0], jnp.float32)

    def _layer_step(h, xs):
        Wlr, blr, att, bias, first = xs
        res = first * res1 + (1.0 - first) * h
        h_next = _gat_layer(h, src_p, dst_p, Wlr, blr, att, bias, res)
        return h_next, 0.0
    h3, _ = lax.scan(_layer_step, x0, (Wlr_s, blr_s, att_s, bias_s, first_s))

    out = _mm(h3, m1_W, m1_b, act="relu")
    out = _mm(out, m2_W, m2_b, act="relu")
    m3_Wp = jnp.pad(m3_W, ((0, 0), (0, _D - 1)))
    m3_bp = jnp.pad(m3_b, (0, _D - 1))
    out = _mm(out, m3_Wp, m3_bp)
    return out[:, 0]


# speedup vs baseline: 1.1974x; 1.1974x over previous
import functools

import jax
import jax.numpy as jnp
from jax import lax
from jax.experimental import pallas as pl
from jax.experimental.pallas import tpu as pltpu
from jax.experimental.pallas import tpu_sc as plsc

_N = 10000
_D = 128
_E = 650000
_NC, _NS, _L = 2, 16, 16
_CHUNK = 96
_BLK = 8
_CPT = 424
_EPAD = _NS * _CHUNK * _CPT
_HALF = 5120
_AROWS = 5248
_RPT = _AROWS // _NS
_XROWS = 10112

_ROWS = 1000


def _mm_body(h_ref, w_ref, b_ref, o_ref, *, act):
    y = jnp.dot(h_ref[...], w_ref[...], preferred_element_type=jnp.float32)
    y = y + b_ref[...][None, :]
    if act == "relu":
        y = jnp.maximum(y, 0.0)
    o_ref[...] = y


def _mm(h, W, b, act=None):
    N, K = h.shape
    M = W.shape[1]
    return pl.pallas_call(
        functools.partial(_mm_body, act=act),
        grid=(N // _ROWS,),
        in_specs=[
            pl.BlockSpec((_ROWS, K), lambda i: (i, 0)),
            pl.BlockSpec((K, M), lambda i: (0, 0)),
            pl.BlockSpec((M,), lambda i: (0,)),
        ],
        out_specs=pl.BlockSpec((_ROWS, M), lambda i: (i, 0)),
        out_shape=jax.ShapeDtypeStruct((N, M), jnp.float32),
    )(h, W, b)


def _combine_body(a_ref, d_ref, bias_ref, res_ref, o_ref):
    den = jnp.sum(d_ref[...], axis=1)
    gat = a_ref[...] / (den + 1e-16)[:, None] + bias_ref[...][None, :]
    o_ref[...] = jnp.maximum(gat, 0.0) + res_ref[...]


def _combine(msg, den, bias, res):
    return pl.pallas_call(
        _combine_body,
        grid=(_N // _ROWS,),
        in_specs=[
            pl.BlockSpec((_ROWS, _D), lambda i: (i, 0)),
            pl.BlockSpec((_ROWS, _NS), lambda i: (i, 0)),
            pl.BlockSpec((_D,), lambda i: (0,)),
            pl.BlockSpec((_ROWS, _D), lambda i: (i, 0)),
        ],
        out_specs=pl.BlockSpec((_ROWS, _D), lambda i: (i, 0)),
        out_shape=jax.ShapeDtypeStruct((_N, _D), jnp.float32),
    )(msg, den, bias, res)


def _edge_body(xl_hbm, xr_hbm, src_hbm, dst_hbm, att_hbm, out_hbm, den_hbm,
               sb_v, db_v, dloc_v, xl0_v, xl1_v, xr0_v, xr1_v,
               stg_v, att_v, den_v, acc_sh, sem0, sem1):
    cid = lax.axis_index("c")
    sid = lax.axis_index("s")

    zeros16 = jnp.zeros((_L,), jnp.float32)
    nbase = cid * _HALF

    xl_b, xr_b = (xl0_v, xl1_v), (xr0_v, xr1_v)
    sem_b = (sem0, sem1)

    def _zero_row(r, carry):
        for j in range(_D // _L):
            stg_v[r, pl.ds(j * _L, _L)] = zeros16
        return carry
    lax.fori_loop(0, _CHUNK, _zero_row, 0)

    def _zero_den(r, carry):
        den_v[pl.ds(r * _L, _L)] = zeros16
        return carry
    lax.fori_loop(0, _AROWS // _L, _zero_den, 0)

    base_r = sid * _RPT
    for k in range(_RPT // _CHUNK):
        pltpu.sync_copy(stg_v, acc_sh.at[pl.ds(base_r + k * _CHUNK, _CHUNK)])
    rem = _RPT % _CHUNK
    if rem:
        pltpu.sync_copy(stg_v.at[pl.ds(0, rem)],
                        acc_sh.at[pl.ds(base_r + (_RPT // _CHUNK) * _CHUNK, rem)])
    plsc.subcore_barrier()

    pltpu.sync_copy(att_hbm, att_v)
    att_r = [att_v[pl.ds(j * _L, _L)] for j in range(_D // _L)]
    iota16 = lax.iota(jnp.int32, _L)

    def _issue(c):
        p = c % 2
        sl = pl.ds(c * _CHUNK, _CHUNK)
        return (
            pltpu.async_copy(xl_hbm.at[sb_v.at[sl]], xl_b[p], sem_b[p]),
            pltpu.async_copy(xr_hbm.at[db_v.at[sl]], xr_b[p], sem_b[p]),
        )

    def _compute(c):
        p = c % 2
        xl_v, xr_v = xl_b[p], xr_b[p]

        def _group(g, c2):
            w_vec = zeros16
            for k in range(_L):
                e = g * _L + k
                terms = []
                for j in range(_D // _L):
                    sl = pl.ds(j * _L, _L)
                    v = xl_v[e, sl] + xr_v[e, sl]
                    terms.append(jnp.maximum(v, 0.2 * v) * att_r[j])
                while len(terms) > 1:
                    terms = [a + b for a, b in zip(terms[::2], terms[1::2])]
                alpha = jnp.sum(terms[0])
                w_spl = jnp.exp(jnp.broadcast_to(alpha, (_L,)))
                for j in range(_D // _L):
                    sl = pl.ds(j * _L, _L)
                    stg_v[e, sl] = xl_v[e, sl] * w_spl
                w_vec = jnp.where(iota16 == k, w_spl, w_vec)
            dl16 = db_v[pl.ds(c * _CHUNK + g * _L, _L)] - nbase
            dl16 = jnp.where((dl16 < 0) | (dl16 >= _HALF), _HALF, dl16)
            dloc_v[pl.ds(g * _L, _L)] = dl16
            plsc.addupdate_scatter(den_v, [dl16], w_vec)
            return c2
        lax.fori_loop(0, _CHUNK // _L, _group, 0)
        pltpu.sync_copy(stg_v, acc_sh.at[dloc_v], add=True)

    def _block(b, carry):
        e0 = (sid * _CPT + b * _BLK) * _CHUNK
        pltpu.sync_copy(src_hbm.at[pl.ds(e0, _BLK * _CHUNK)], sb_v)
        pltpu.sync_copy(dst_hbm.at[pl.ds(e0, _BLK * _CHUNK)], db_v)
        cps = _issue(0)
        for c in range(_BLK):
            nxt = _issue(c + 1) if c + 1 < _BLK else None
            for cp in cps:
                cp.wait()
            _compute(c)
            cps = nxt
        return carry
    lax.fori_loop(0, _CPT // _BLK, _block, 0)

    pltpu.sync_copy(den_v, den_hbm.at[cid, sid])
    plsc.subcore_barrier()
    pltpu.sync_copy(acc_sh.at[pl.ds(base_r, _RPT)],
                    out_hbm.at[cid, pl.ds(base_r, _RPT)])


_edge_kernel = pl.kernel(
    _edge_body,
    out_type=(
        jax.ShapeDtypeStruct((_NC, _AROWS, _D), jnp.float32),
        jax.ShapeDtypeStruct((_NC, _NS, _AROWS), jnp.float32),
    ),
    mesh=plsc.VectorSubcoreMesh(core_axis_name="c", subcore_axis_name="s",
                                num_cores=_NC, num_subcores=_NS),
    scratch_types=[
        pltpu.VMEM((_BLK * _CHUNK,), jnp.int32),
        pltpu.VMEM((_BLK * _CHUNK,), jnp.int32),
        pltpu.VMEM((_CHUNK,), jnp.int32),
        pltpu.VMEM((_CHUNK, _D), jnp.float32),
        pltpu.VMEM((_CHUNK, _D), jnp.float32),
        pltpu.VMEM((_CHUNK, _D), jnp.float32),
        pltpu.VMEM((_CHUNK, _D), jnp.float32),
        pltpu.VMEM((_CHUNK, _D), jnp.float32),
        pltpu.VMEM((_D,), jnp.float32),
        pltpu.VMEM((_AROWS,), jnp.float32),
        pltpu.VMEM_SHARED((_AROWS, _D), jnp.float32),
        pltpu.SemaphoreType.DMA,
        pltpu.SemaphoreType.DMA,
    ],
    compiler_params=pltpu.CompilerParams(needs_layout_passes=False),
)


def _gat_layer(h, src_p, dst_p, Wlr, blr, att, bias, res):
    xlr = _mm(h, Wlr, blr)
    xl = xlr[:, :_D]
    xr = jnp.pad(xlr[:, _D:], ((0, _XROWS - _N), (0, 0)))
    acc, den = _edge_kernel(xl, xr, src_p, dst_p, att)
    msg = jnp.concatenate([acc[0, :_HALF], acc[1, :_N - _HALF]], axis=0)
    den_t = jnp.concatenate(
        [
            den[0].T[:_HALF],
            den[1].T[:_N - _HALF],
        ],
        axis=0,
    )
    return _combine(msg, den_t, bias, res)


def kernel(x, edge_index, W_in, b_in, c1_Wl, c1_bl, c1_Wr, c1_br, c1_att, c1_bias, skip_W, skip_b, c2_Wl, c2_bl, c2_Wr, c2_br, c2_att, c2_bias, c3_Wl, c3_bl, c3_Wr, c3_br, c3_att, c3_bias, m1_W, m1_b, m2_W, m2_b, m3_W, m3_b):
    loop = jnp.arange(_N, dtype=edge_index.dtype)
    pad = _EPAD - _E
    src_p = jnp.concatenate([edge_index[0], loop, jnp.zeros((pad,), jnp.int32)])
    dst_p = jnp.concatenate([edge_index[1], loop, jnp.full((pad,), _N, jnp.int32)])

    x0 = _mm(x, W_in, b_in, act="relu")
    res1 = _mm(x0, skip_W, skip_b)

    Wlr_s = jnp.stack([jnp.concatenate([Wl, Wr], axis=1)
                       for Wl, Wr in ((c1_Wl, c1_Wr), (c2_Wl, c2_Wr), (c3_Wl, c3_Wr))])
    blr_s = jnp.stack([jnp.concatenate([bl, br], axis=0)
                       for bl, br in ((c1_bl, c1_br), (c2_bl, c2_br), (c3_bl, c3_br))])
    att_s = jnp.stack([c1_att[0], c2_att[0], c3_att[0]])
    bias_s = jnp.stack([c1_bias, c2_bias, c3_bias])
    first_s = jnp.array([1.0, 0.0, 0.0], jnp.float32)

    def _layer_step(h, xs):
        Wlr, blr, att, bias, first = xs
        res = first * res1 + (1.0 - first) * h
        h_next = _gat_layer(h, src_p, dst_p, Wlr, blr, att, bias, res)
        return h_next, 0.0
    h3, _ = lax.scan(_layer_step, x0, (Wlr_s, blr_s, att_s, bias_s, first_s))

    out = _mm(h3, m1_W, m1_b, act="relu")
    out = _mm(out, m2_W, m2_b, act="relu")
    m3_Wp = jnp.pad(m3_W, ((0, 0), (0, _D - 1)))
    m3_bp = jnp.pad(m3_b, (0, _D - 1))
    out = _mm(out, m3_Wp, m3_bp)
    return out[:, 0]
